# Initial kernel scaffold; baseline (speedup 1.0000x reference)
import jax
import jax.numpy as jnp
from jax.experimental import pallas as pl


def kernel(inputs, weight):
    raise NotImplementedError("write your pallas kernel here")


# speedup vs baseline: 1.3545x; 1.3545x over previous
import functools

import jax
import jax.numpy as jnp
from jax import lax
from jax.experimental import pallas as pl
from jax.experimental.pallas import tpu as pltpu

_COMMITMENT_COST = 0.25
_BLOCK = 2048


def _vq_block(x_ref, xsq_ref, wsq_ref, wt_ref, w_ref, q_ref, loss_ref):
    x = x_ref[...]
    wt = wt_ref[...]
    dot = jnp.dot(x, wt, preferred_element_type=jnp.float32)
    d = (xsq_ref[...] + wsq_ref[...]) - 2.0 * dot
    m = jnp.min(d, axis=1, keepdims=True)
    cols = lax.broadcasted_iota(jnp.int32, d.shape, 1)
    idx = jnp.min(jnp.where(d == m, cols, d.shape[1]), axis=1, keepdims=True)
    onehot = (cols == idx).astype(jnp.float32)
    q = jax.lax.dot(onehot, w_ref[...],
                    precision=jax.lax.Precision.HIGHEST)
    q_ref[...] = q
    r = q - x
    part = jnp.sum(r * r)

    @pl.when(pl.program_id(0) == 0)
    def _init():
        loss_ref[0, 0] = 0.0

    loss_ref[0, 0] += part


@functools.partial(jax.jit, static_argnames=())
def _vq_tc(flat, x_sq, w_sq, w_t, weight):
    n, d = flat.shape
    k = weight.shape[0]
    grid = n // _BLOCK
    q, loss = pl.pallas_call(
        _vq_block,
        grid=(grid,),
        in_specs=[
            pl.BlockSpec((_BLOCK, d), lambda i: (i, 0)),
            pl.BlockSpec((_BLOCK, 1), lambda i: (i, 0)),
            pl.BlockSpec((1, k), lambda i: (0, 0)),
            pl.BlockSpec((d, k), lambda i: (0, 0)),
            pl.BlockSpec((k, d), lambda i: (0, 0)),
        ],
        out_specs=[
            pl.BlockSpec((_BLOCK, d), lambda i: (i, 0)),
            pl.BlockSpec((1, 1), lambda i: (0, 0), memory_space=pltpu.SMEM),
        ],
        out_shape=[
            jax.ShapeDtypeStruct((n, d), jnp.float32),
            jax.ShapeDtypeStruct((1, 1), jnp.float32),
        ],
    )(flat, x_sq, w_sq, w_t, weight)
    return q, loss


def kernel(inputs, weight):
    d = weight.shape[1]
    flat = inputs.reshape(-1, d)
    x_sq = jnp.sum(flat ** 2, axis=1, keepdims=True)
    w_sq = jnp.sum(weight ** 2, axis=1).reshape(1, -1)
    q, loss_sum = _vq_tc(flat, x_sq, w_sq, weight.T, weight)
    mean_sq = loss_sum[0, 0] / jnp.float32(flat.size)
    quantization_loss = mean_sq
    commitment_loss = _COMMITMENT_COST * mean_sq
    return (q.reshape(inputs.shape), quantization_loss, commitment_loss)


# speedup vs baseline: 1.5518x; 1.1457x over previous
import functools

import jax
import jax.numpy as jnp
from jax import lax
from jax.experimental import pallas as pl
from jax.experimental.pallas import tpu as pltpu
from jax.experimental.pallas import tpu_sc as plsc

_COMMITMENT_COST = 0.25
_BLOCK = 2048
_NC, _NS, _L = 2, 16, 16
_NW = _NC * _NS


def _vq_argmin_block(x_ref, wsq_ref, wt_ref, idx_ref, loss_ref):
    x = x_ref[...]
    wt = wt_ref[...]
    x_sq = jnp.sum(x * x, axis=1, keepdims=True)
    dot = jnp.dot(x, wt, preferred_element_type=jnp.float32)
    d = (x_sq + wsq_ref[...]) - 2.0 * dot
    m = jnp.min(d, axis=1, keepdims=True)
    cols = lax.broadcasted_iota(jnp.int32, d.shape, 1)
    idx = jnp.min(jnp.where(d == m, cols, d.shape[1]), axis=1)
    idx_ref[...] = idx.reshape(idx_ref.shape)

    @pl.when(pl.program_id(0) == 0)
    def _init():
        loss_ref[0, 0] = 0.0

    loss_ref[0, 0] += jnp.sum(m)


def _tc_argmin(flat, w_sq, w_t):
    n, d = flat.shape
    k = w_t.shape[1]
    grid = n // _BLOCK
    rows = _BLOCK // 128
    return pl.pallas_call(
        _vq_argmin_block,
        grid=(grid,),
        in_specs=[
            pl.BlockSpec((_BLOCK, d), lambda i: (i, 0)),
            pl.BlockSpec((1, k), lambda i: (0, 0)),
            pl.BlockSpec((d, k), lambda i: (0, 0)),
        ],
        out_specs=[
            pl.BlockSpec((rows, 128), lambda i: (i, 0)),
            pl.BlockSpec((1, 1), lambda i: (0, 0), memory_space=pltpu.SMEM),
        ],
        out_shape=[
            jax.ShapeDtypeStruct((n // 128, 128), jnp.int32),
            jax.ShapeDtypeStruct((1, 1), jnp.float32),
        ],
    )(flat, w_sq, w_t)


def _sc_gather(w_flat, idx):
    n = idx.shape[0]
    d = 32
    kd = w_flat.shape[0]
    b_per_w = n // _NW
    groups = b_per_w // _L
    mesh = plsc.VectorSubcoreMesh(core_axis_name="c", subcore_axis_name="s")

    @functools.partial(
        pl.kernel, mesh=mesh,
        compiler_params=pltpu.CompilerParams(needs_layout_passes=False),
        out_type=jax.ShapeDtypeStruct((n * d,), jnp.float32),
        scratch_types=[
            pltpu.VMEM((kd,), jnp.float32),
            pltpu.VMEM((b_per_w,), jnp.int32),
            pltpu.VMEM((b_per_w * d,), jnp.float32),
        ],
    )
    def k(w_hbm, idx_hbm, out_hbm, w_v, idx_v, rows_v):
        wid = lax.axis_index("s") * _NC + lax.axis_index("c")
        base = wid * b_per_w
        pltpu.sync_copy(w_hbm, w_v)
        pltpu.sync_copy(idx_hbm.at[pl.ds(base, b_per_w)], idx_v)
        lane = lax.iota(jnp.int32, _L)

        def body(g, carry):
            iv = idx_v[pl.ds(g * _L, _L)]
            src = iv * d
            rowbase = g * (_L * d) + lane * d
            for c in range(d):
                vals = plsc.load_gather(w_v, [src + c])
                plsc.store_scatter(rows_v, [rowbase + c], vals)
            return carry

        lax.fori_loop(0, groups, body, 0)
        pltpu.sync_copy(rows_v, out_hbm.at[pl.ds(base * d, b_per_w * d)])

    return k(w_flat, idx)


@jax.jit
def _vq(inputs, weight):
    d = weight.shape[1]
    flat = inputs.reshape(-1, d)
    w_sq = jnp.sum(weight ** 2, axis=1).reshape(1, -1)
    idx, loss_sum = _tc_argmin(flat, w_sq, weight.T)
    q = _sc_gather(weight.reshape(-1), idx.reshape(-1))
    mean_sq = loss_sum[0, 0] / jnp.float32(flat.size)
    return (q.reshape(inputs.shape), mean_sq, _COMMITMENT_COST * mean_sq)


def kernel(inputs, weight):
    return _vq(inputs, weight)


# speedup vs baseline: 2.1538x; 1.3879x over previous
import functools

import jax
import jax.numpy as jnp
from jax import lax
from jax.experimental import pallas as pl
from jax.experimental.pallas import tpu as pltpu
from jax.experimental.pallas import tpu_sc as plsc

_COMMITMENT_COST = 0.25
_BLOCK = 2048
_NC, _NS, _L = 2, 16, 16
_NW = _NC * _NS


def _vq_argmin_block(x_ref, wsq_ref, wt_ref, idx_ref, loss_ref):
    x = x_ref[...]
    wt = wt_ref[...]
    x_sq = jnp.sum(x * x, axis=1, keepdims=True)
    dot = jnp.dot(x, wt, preferred_element_type=jnp.float32)
    d = (x_sq + wsq_ref[...]) - 2.0 * dot
    m = jnp.min(d, axis=1, keepdims=True)
    cols = lax.broadcasted_iota(jnp.int32, d.shape, 1)
    idx = jnp.min(jnp.where(d == m, cols, d.shape[1]), axis=1)
    idx_ref[...] = idx.reshape(idx_ref.shape)

    @pl.when(pl.program_id(0) == 0)
    def _init():
        loss_ref[0, 0] = 0.0

    loss_ref[0, 0] += jnp.sum(m)


def _tc_argmin(flat, w_sq, w_t):
    n, d = flat.shape
    k = w_t.shape[1]
    grid = n // _BLOCK
    rows = _BLOCK // 128
    return pl.pallas_call(
        _vq_argmin_block,
        grid=(grid,),
        in_specs=[
            pl.BlockSpec((_BLOCK, d), lambda i: (i, 0)),
            pl.BlockSpec((1, k), lambda i: (0, 0)),
            pl.BlockSpec((d, k), lambda i: (0, 0)),
        ],
        out_specs=[
            pl.BlockSpec((rows, 128), lambda i: (i, 0)),
            pl.BlockSpec((1, 1), lambda i: (0, 0), memory_space=pltpu.SMEM),
        ],
        out_shape=[
            jax.ShapeDtypeStruct((n // 128, 128), jnp.int32),
            jax.ShapeDtypeStruct((1, 1), jnp.float32),
        ],
    )(flat, w_sq, w_t)


def _sc_gather(weight, idx):
    n = idx.shape[0]
    d = weight.shape[1]
    b_per_w = n // _NW
    mesh = plsc.VectorSubcoreMesh(core_axis_name="c", subcore_axis_name="s")

    @functools.partial(
        pl.kernel, mesh=mesh,
        compiler_params=pltpu.CompilerParams(
            needs_layout_passes=False, use_tc_tiling_on_sc=False),
        out_type=jax.ShapeDtypeStruct((n, d), jnp.float32),
        scratch_types=[
            pltpu.VMEM((b_per_w,), jnp.int32),
            pltpu.VMEM((b_per_w, d), jnp.float32),
            pltpu.SemaphoreType.DMA,
        ],
    )
    def k(table_hbm, idx_hbm, out_hbm, idx_v, rows_v, sem):
        wid = lax.axis_index("s") * _NC + lax.axis_index("c")
        base = wid * b_per_w
        pltpu.sync_copy(idx_hbm.at[pl.ds(base, b_per_w)], idx_v)
        pltpu.async_copy(table_hbm.at[idx_v], rows_v, sem).wait()
        pltpu.sync_copy(rows_v, out_hbm.at[pl.ds(base, b_per_w)])

    return k(weight, idx)


@jax.jit
def _vq(inputs, weight):
    d = weight.shape[1]
    flat = inputs.reshape(-1, d)
    w_sq = jnp.sum(weight ** 2, axis=1).reshape(1, -1)
    idx, loss_sum = _tc_argmin(flat, w_sq, weight.T)
    q = _sc_gather(weight, idx.reshape(-1))
    mean_sq = loss_sum[0, 0] / jnp.float32(flat.size)
    return (q.reshape(inputs.shape), mean_sq, _COMMITMENT_COST * mean_sq)


def kernel(inputs, weight):
    return _vq(inputs, weight)


# speedup vs baseline: 2.1831x; 1.0136x over previous
import functools

import jax
import jax.numpy as jnp
from jax import lax
from jax.experimental import pallas as pl
from jax.experimental.pallas import tpu as pltpu
from jax.experimental.pallas import tpu_sc as plsc

_COMMITMENT_COST = 0.25
_BLOCK = 4096
_NC, _NS, _L = 2, 16, 16
_NW = _NC * _NS


def _vq_argmin_block(x_ref, wsq_ref, wt_ref, idx_ref, loss_ref):
    x = x_ref[...]
    wt = wt_ref[...]
    x_sq = jnp.sum(x * x, axis=1, keepdims=True)
    dot = jnp.dot(x, wt, preferred_element_type=jnp.float32)
    d = (x_sq + wsq_ref[...]) - 2.0 * dot
    m = jnp.min(d, axis=1, keepdims=True)
    cols = lax.broadcasted_iota(jnp.int32, d.shape, 1)
    idx = jnp.min(jnp.where(d == m, cols, d.shape[1]), axis=1)
    idx_ref[...] = idx.reshape(idx_ref.shape)

    @pl.when(pl.program_id(0) == 0)
    def _init():
        loss_ref[0, 0] = 0.0

    loss_ref[0, 0] += jnp.sum(m)


def _tc_argmin(flat, w_sq, w_t):
    n, d = flat.shape
    k = w_t.shape[1]
    grid = n // _BLOCK
    rows = _BLOCK // 128
    return pl.pallas_call(
        _vq_argmin_block,
        grid=(grid,),
        in_specs=[
            pl.BlockSpec((_BLOCK, d), lambda i: (i, 0)),
            pl.BlockSpec((1, k), lambda i: (0, 0)),
            pl.BlockSpec((d, k), lambda i: (0, 0)),
        ],
        out_specs=[
            pl.BlockSpec((rows, 128), lambda i: (i, 0)),
            pl.BlockSpec((1, 1), lambda i: (0, 0), memory_space=pltpu.SMEM),
        ],
        out_shape=[
            jax.ShapeDtypeStruct((n // 128, 128), jnp.int32),
            jax.ShapeDtypeStruct((1, 1), jnp.float32),
        ],
    )(flat, w_sq, w_t)


def _sc_gather(weight, idx):
    n = idx.shape[0]
    d = weight.shape[1]
    b_per_w = n // _NW
    mesh = plsc.VectorSubcoreMesh(core_axis_name="c", subcore_axis_name="s")

    @functools.partial(
        pl.kernel, mesh=mesh,
        compiler_params=pltpu.CompilerParams(
            needs_layout_passes=False, use_tc_tiling_on_sc=False),
        out_type=jax.ShapeDtypeStruct((n, d), jnp.float32),
        scratch_types=[
            pltpu.VMEM((b_per_w,), jnp.int32),
            pltpu.VMEM((b_per_w, d), jnp.float32),
            pltpu.SemaphoreType.DMA,
        ],
    )
    def k(table_hbm, idx_hbm, out_hbm, idx_v, rows_v, sem):
        wid = lax.axis_index("s") * _NC + lax.axis_index("c")
        base = wid * b_per_w
        pltpu.sync_copy(idx_hbm.at[pl.ds(base, b_per_w)], idx_v)
        pltpu.async_copy(table_hbm.at[idx_v], rows_v, sem).wait()
        pltpu.sync_copy(rows_v, out_hbm.at[pl.ds(base, b_per_w)])

    return k(weight, idx)


@jax.jit
def _vq(inputs, weight):
    d = weight.shape[1]
    flat = inputs.reshape(-1, d)
    w_sq = jnp.sum(weight ** 2, axis=1).reshape(1, -1)
    idx, loss_sum = _tc_argmin(flat, w_sq, weight.T)
    q = _sc_gather(weight, idx.reshape(-1))
    mean_sq = loss_sum[0, 0] / jnp.float32(flat.size)
    return (q.reshape(inputs.shape), mean_sq, _COMMITMENT_COST * mean_sq)


def kernel(inputs, weight):
    return _vq(inputs, weight)
